# baseline (speedup 1.0000x reference)
import functools

import jax
import jax.numpy as jnp
from jax.experimental import pallas as pl
from jax.experimental.pallas import tpu as pltpu

NUM_KEEP = 8192


def _score_body(feat_ref, coord_ref, w1f_ref, w1c_ref, b1_ref, w2_ref, out_ref):
    h = jnp.dot(feat_ref[...], w1f_ref[...], preferred_element_type=jnp.float32)
    h = h + jnp.dot(coord_ref[...], w1c_ref[...], preferred_element_type=jnp.float32)
    h = h + b1_ref[...]
    h = jnp.maximum(h, 0.0)
    out_ref[...] = jnp.dot(h, w2_ref[...], preferred_element_type=jnp.float32)


def _scores(feat, coord, W1, b1, W2):
    N, D = feat.shape
    blk = min(1024, N)
    grid = N // blk
    w1f = W1[:D]
    w1c = W1[D:]
    w2p = jnp.pad(W2, ((0, 0), (0, 7)))
    b1r = b1.reshape(1, D)
    out = pl.pallas_call(
        _score_body,
        grid=(grid,),
        in_specs=[
            pl.BlockSpec((blk, D), lambda i: (i, 0)),
            pl.BlockSpec((blk, 3), lambda i: (i, 0)),
            pl.BlockSpec((D, D), lambda i: (0, 0)),
            pl.BlockSpec((3, D), lambda i: (0, 0)),
            pl.BlockSpec((1, D), lambda i: (0, 0)),
            pl.BlockSpec((D, 8), lambda i: (0, 0)),
        ],
        out_specs=pl.BlockSpec((blk, 8), lambda i: (i, 0)),
        out_shape=jax.ShapeDtypeStruct((N, 8), jnp.float32),
    )(feat, coord, w1f, w1c, b1r, w2p)
    return out[:, 0]


def kernel(feat_list, coord_list, W1, b1, W2, b2):
    B, N, D = feat_list.shape
    M = min(NUM_KEEP, N)
    feat = feat_list[0]
    coord = coord_list[0]
    scores = _scores(feat, coord, W1, b1, W2)
    _, idx = jax.lax.top_k(scores, M)
    feats = jnp.take(feat, idx, axis=0)
    coords = jnp.take(coord, idx, axis=0)
    return feats[None], coords[None]
